# Initial kernel scaffold; baseline (speedup 1.0000x reference)
import jax
import jax.numpy as jnp
from jax.experimental import pallas as pl


def kernel(x, ins_indices_batch, ins_ids_list, gamma, beta):
    raise NotImplementedError("write your pallas kernel here")


# speedup vs baseline: 16.4607x; 16.4607x over previous
import jax
import jax.numpy as jnp
from jax.experimental import pallas as pl
from jax.experimental.pallas import tpu as pltpu

N, C, H, W = 4, 96, 384, 384
G = 32
CG = C // G
EPS = 1e-5
NUM_IDS = 8
BH = 16


def _gn_kernel(ids_ref, x_ref, idx_ref, gamma_ref, beta_ref, out_ref):
    n = pl.program_id(0)
    xb = x_ref[0]
    xg = xb.reshape(G, CG, BH, W)
    mean = jnp.mean(xg, axis=1, keepdims=True)
    diff = xg - mean
    var = jnp.mean(diff * diff, axis=1, keepdims=True)
    xnorm = (diff * jax.lax.rsqrt(var + EPS)).reshape(C, BH, W)
    gamma = gamma_ref[...][:, :, None]
    beta = beta_ref[...][:, :, None]
    xnorm = xnorm * gamma + beta
    idxb = idx_ref[0]
    mask = idxb == ids_ref[n, 0]
    for i in range(1, NUM_IDS):
        mask = mask | (idxb == ids_ref[n, i])
    out_ref[0] = jnp.where(mask[None, :, :], xnorm, xb)


def kernel(x, ins_indices_batch, ins_ids_list, gamma, beta):
    gamma2 = gamma.reshape(C, 1)
    beta2 = beta.reshape(C, 1)
    grid = (N, H // BH)
    out = pl.pallas_call(
        _gn_kernel,
        grid=grid,
        in_specs=[
            pl.BlockSpec(memory_space=pltpu.SMEM),
            pl.BlockSpec((1, C, BH, W), lambda n, h: (n, 0, h, 0)),
            pl.BlockSpec((1, BH, W), lambda n, h: (n, h, 0)),
            pl.BlockSpec((C, 1), lambda n, h: (0, 0)),
            pl.BlockSpec((C, 1), lambda n, h: (0, 0)),
        ],
        out_specs=pl.BlockSpec((1, C, BH, W), lambda n, h: (n, 0, h, 0)),
        out_shape=jax.ShapeDtypeStruct((N, C, H, W), x.dtype),
    )(ins_ids_list, x, ins_indices_batch, gamma2, beta2)
    return out


# speedup vs baseline: 19.6152x; 1.1916x over previous
import jax
import jax.numpy as jnp
from jax.experimental import pallas as pl
from jax.experimental.pallas import tpu as pltpu

N, C, H, W = 4, 96, 384, 384
G = 32
CG = C // G
EPS = 1e-5
NUM_IDS = 8
BH = 48


def _gn_kernel(ids_ref, x_ref, idx_ref, gamma_ref, beta_ref, out_ref):
    n = pl.program_id(0)
    xb = x_ref[0]
    xg = xb.reshape(G, CG, BH, W)
    mean = jnp.mean(xg, axis=1, keepdims=True)
    diff = xg - mean
    var = jnp.mean(diff * diff, axis=1, keepdims=True)
    xnorm = (diff * jax.lax.rsqrt(var + EPS)).reshape(C, BH, W)
    gamma = gamma_ref[...][:, :, None]
    beta = beta_ref[...][:, :, None]
    xnorm = xnorm * gamma + beta
    idxb = idx_ref[0]
    mask = idxb == ids_ref[n, 0]
    for i in range(1, NUM_IDS):
        mask = mask | (idxb == ids_ref[n, i])
    out_ref[0] = jnp.where(mask[None, :, :], xnorm, xb)


def kernel(x, ins_indices_batch, ins_ids_list, gamma, beta):
    gamma2 = gamma.reshape(C, 1)
    beta2 = beta.reshape(C, 1)
    grid = (N, H // BH)
    out = pl.pallas_call(
        _gn_kernel,
        grid=grid,
        in_specs=[
            pl.BlockSpec(memory_space=pltpu.SMEM),
            pl.BlockSpec((1, C, BH, W), lambda n, h: (n, 0, h, 0)),
            pl.BlockSpec((1, BH, W), lambda n, h: (n, h, 0)),
            pl.BlockSpec((C, 1), lambda n, h: (0, 0)),
            pl.BlockSpec((C, 1), lambda n, h: (0, 0)),
        ],
        out_specs=pl.BlockSpec((1, C, BH, W), lambda n, h: (n, 0, h, 0)),
        out_shape=jax.ShapeDtypeStruct((N, C, H, W), x.dtype),
    )(ins_ids_list, x, ins_indices_batch, gamma2, beta2)
    return out
